# Initial kernel scaffold; baseline (speedup 1.0000x reference)
import functools

import jax
import jax.numpy as jnp
from jax import lax
from jax.experimental import pallas as pl
from jax.experimental.pallas import tpu as pltpu
from jax.experimental.pallas import tpu_sc as plsc

N_NODES = 10000
D = 128
N_EDGES = 320000

NC = 2
NS = 16
NW = NC * NS

CHUNK = 128
CHUNKS_PER_W = 80
EDGES_PER_W = CHUNK * CHUNKS_PER_W
E_PAD = NW * EDGES_PER_W
ROWS_PER_TILE = N_NODES // NS


def _matmul_body(x_ref, wt_ref, o_ref):
    o_ref[...] = jnp.dot(x_ref[...], wt_ref[...],
                         preferred_element_type=jnp.float32)


def _combine_body(a_ref, b_ref, alpha_ref, o_ref):
    s = a_ref[...] + b_ref[...]
    o_ref[...] = jnp.where(s >= 0, s, alpha_ref[0, 0] * s)


def _sc_scatter_body(fts_hbm, src_hbm, dst_hbm, w_hbm, out_hbm,
                     src_v, dst_v, w_v, rows, acc, gsem):
    cid = lax.axis_index("c")
    sid = lax.axis_index("s")
    wid = sid * NC + cid

    pltpu.sync_copy(src_hbm.at[wid], src_v)
    pltpu.sync_copy(dst_hbm.at[wid], dst_v)
    pltpu.sync_copy(w_hbm.at[wid], w_v)

    z = jnp.zeros((16,), jnp.float32)

    def zrow(r, carry):
        for c in range(8):
            rows[r, pl.ds(16 * c, 16)] = z
        return carry

    lax.fori_loop(0, 125, zrow, 0)
    base = sid * ROWS_PER_TILE
    for i in range(5):
        pltpu.sync_copy(rows.at[pl.ds(0, 125)],
                        acc.at[pl.ds(base + i * 125, 125)])
    plsc.subcore_barrier()

    def chunk(j, carry):
        pltpu.async_copy(fts_hbm.at[src_v.at[j]], rows, gsem).wait()

        def mrow(r, c2):
            ws = w_v[j, r]
            for c in range(8):
                sl = rows[r, pl.ds(16 * c, 16)]
                rows[r, pl.ds(16 * c, 16)] = sl * ws
            return c2

        lax.fori_loop(0, CHUNK, mrow, 0)

        pltpu.sync_copy(rows, acc.at[dst_v.at[j]], add=True)
        return carry

    lax.fori_loop(0, CHUNKS_PER_W, chunk, 0)
    plsc.subcore_barrier()

    pltpu.sync_copy(acc.at[pl.ds(base, ROWS_PER_TILE)],
                    out_hbm.at[cid, pl.ds(base, ROWS_PER_TILE)])


_sc_scatter = functools.partial(
    pl.kernel,
    out_type=jax.ShapeDtypeStruct((NC, N_NODES, D), jnp.float32),
    mesh=plsc.VectorSubcoreMesh(core_axis_name="c", subcore_axis_name="s"),
    scratch_types=[
        pltpu.VMEM((CHUNKS_PER_W, CHUNK), jnp.int32),
        pltpu.VMEM((CHUNKS_PER_W, CHUNK), jnp.int32),
        pltpu.VMEM((CHUNKS_PER_W, CHUNK), jnp.float32),
        pltpu.VMEM((CHUNK, D), jnp.float32),
        pltpu.VMEM_SHARED((N_NODES, D), jnp.float32),
        pltpu.SemaphoreType.DMA,
    ],
)(_sc_scatter_body)


def kernel(seq, edge_index, edge_weight, W, alpha):
    wt = W.T
    fts = pl.pallas_call(
        _matmul_body,
        grid=(10,),
        in_specs=[
            pl.BlockSpec((N_NODES // 10, D), lambda i: (i, 0)),
            pl.BlockSpec((D, D), lambda i: (0, 0)),
        ],
        out_specs=pl.BlockSpec((N_NODES // 10, D), lambda i: (i, 0)),
        out_shape=jax.ShapeDtypeStruct((N_NODES, D), jnp.float32),
    )(seq, wt)

    pad = E_PAD - N_EDGES
    pad_rows = (jnp.arange(pad, dtype=jnp.int32) % N_NODES)
    dst = jnp.concatenate([edge_index[0], pad_rows])
    src = jnp.concatenate([edge_index[1], pad_rows])
    w = jnp.concatenate([edge_weight, jnp.zeros((pad,), jnp.float32)])
    src3 = src.reshape(NW, CHUNKS_PER_W, CHUNK)
    dst3 = dst.reshape(NW, CHUNKS_PER_W, CHUNK)
    w3 = w.reshape(NW, CHUNKS_PER_W, CHUNK)

    partials = _sc_scatter(fts, src3, dst3, w3)

    alpha2 = jnp.asarray(alpha, jnp.float32).reshape(1, 1)
    out = pl.pallas_call(
        _combine_body,
        grid=(10,),
        in_specs=[
            pl.BlockSpec((N_NODES // 10, D), lambda i: (i, 0)),
            pl.BlockSpec((N_NODES // 10, D), lambda i: (i, 0)),
            pl.BlockSpec((1, 1), lambda i: (0, 0)),
        ],
        out_specs=pl.BlockSpec((N_NODES // 10, D), lambda i: (i, 0)),
        out_shape=jax.ShapeDtypeStruct((N_NODES, D), jnp.float32),
    )(partials[0], partials[1], alpha2)
    return out


# speedup vs baseline: 7.1070x; 7.1070x over previous
import functools

import jax
import jax.numpy as jnp
from jax import lax
from jax.experimental import pallas as pl
from jax.experimental.pallas import tpu as pltpu
from jax.experimental.pallas import tpu_sc as plsc

N_NODES = 10000
D = 128
N_EDGES = 320000

NC = 2
NS = 16
NW = NC * NS

CHUNK = 128
CHUNKS_PER_W = 80
EDGES_PER_W = CHUNK * CHUNKS_PER_W
E_PAD = NW * EDGES_PER_W
N_PAD = 10240
ROWS_PER_TILE = N_PAD // NS


def _matmul_body(x_ref, wt_ref, o_ref):
    o_ref[...] = jnp.dot(x_ref[...], wt_ref[...],
                         preferred_element_type=jnp.float32)


def _combine_body(a_ref, b_ref, alpha_ref, o_ref):
    s = a_ref[...] + b_ref[...]
    o_ref[...] = jnp.where(s >= 0, s, alpha_ref[0, 0] * s)


def _sc_scatter_body(fts_hbm, src_hbm, dst_hbm, w_hbm, out_hbm,
                     src_v, dst_v, w_v, rows, acc, gsem):
    cid = lax.axis_index("c")
    sid = lax.axis_index("s")
    wid = sid * NC + cid

    pltpu.sync_copy(src_hbm.at[wid], src_v)
    pltpu.sync_copy(dst_hbm.at[wid], dst_v)
    pltpu.sync_copy(w_hbm.at[wid], w_v)

    z = jnp.zeros((16,), jnp.float32)

    def zrow(r, carry):
        for c in range(8):
            rows[r, pl.ds(16 * c, 16)] = z
        return carry

    lax.fori_loop(0, CHUNK, zrow, 0)
    base = sid * ROWS_PER_TILE
    for i in range(5):
        pltpu.sync_copy(rows,
                        acc.at[pl.ds(base + i * CHUNK, CHUNK)])
    plsc.subcore_barrier()

    def chunk(j, carry):
        pltpu.async_copy(fts_hbm.at[src_v.at[j]], rows, gsem).wait()

        def mgroup(g, c2):
            wvec = w_v[j, pl.ds(16 * g, 16)]
            for l in range(16):
                ws = wvec[l]
                r = 16 * g + l
                for c in range(8):
                    sl = rows[r, pl.ds(16 * c, 16)]
                    rows[r, pl.ds(16 * c, 16)] = sl * ws
            return c2

        lax.fori_loop(0, CHUNK // 16, mgroup, 0)

        pltpu.sync_copy(rows, acc.at[dst_v.at[j]], add=True)
        return carry

    lax.fori_loop(0, CHUNKS_PER_W, chunk, 0)
    plsc.subcore_barrier()

    pltpu.sync_copy(acc.at[pl.ds(base, ROWS_PER_TILE)],
                    out_hbm.at[cid, pl.ds(base, ROWS_PER_TILE)])


_sc_scatter = functools.partial(
    pl.kernel,
    out_type=jax.ShapeDtypeStruct((NC, N_PAD, D), jnp.float32),
    mesh=plsc.VectorSubcoreMesh(core_axis_name="c", subcore_axis_name="s"),
    scratch_types=[
        pltpu.VMEM((CHUNKS_PER_W, CHUNK), jnp.int32),
        pltpu.VMEM((CHUNKS_PER_W, CHUNK), jnp.int32),
        pltpu.VMEM((CHUNKS_PER_W, CHUNK), jnp.float32),
        pltpu.VMEM((CHUNK, D), jnp.float32),
        pltpu.VMEM_SHARED((N_PAD, D), jnp.float32),
        pltpu.SemaphoreType.DMA,
    ],
)(_sc_scatter_body)


def kernel(seq, edge_index, edge_weight, W, alpha):
    wt = W.T
    fts = pl.pallas_call(
        _matmul_body,
        grid=(10,),
        in_specs=[
            pl.BlockSpec((N_NODES // 10, D), lambda i: (i, 0)),
            pl.BlockSpec((D, D), lambda i: (0, 0)),
        ],
        out_specs=pl.BlockSpec((N_NODES // 10, D), lambda i: (i, 0)),
        out_shape=jax.ShapeDtypeStruct((N_NODES, D), jnp.float32),
    )(seq, wt)

    pad = E_PAD - N_EDGES
    pad_rows = (jnp.arange(pad, dtype=jnp.int32) % N_NODES)
    dst = jnp.concatenate([edge_index[0], pad_rows])
    src = jnp.concatenate([edge_index[1], pad_rows])
    w = jnp.concatenate([edge_weight, jnp.zeros((pad,), jnp.float32)])
    src3 = src.reshape(NW, CHUNKS_PER_W, CHUNK)
    dst3 = dst.reshape(NW, CHUNKS_PER_W, CHUNK)
    w3 = w.reshape(NW, CHUNKS_PER_W, CHUNK)

    partials = _sc_scatter(fts, src3, dst3, w3)

    alpha2 = jnp.asarray(alpha, jnp.float32).reshape(1, 1)
    out = pl.pallas_call(
        _combine_body,
        grid=(10,),
        in_specs=[
            pl.BlockSpec((N_NODES // 10, D), lambda i: (i, 0)),
            pl.BlockSpec((N_NODES // 10, D), lambda i: (i, 0)),
            pl.BlockSpec((1, 1), lambda i: (0, 0)),
        ],
        out_specs=pl.BlockSpec((N_NODES // 10, D), lambda i: (i, 0)),
        out_shape=jax.ShapeDtypeStruct((N_NODES, D), jnp.float32),
    )(partials[0], partials[1], alpha2)
    return out


# speedup vs baseline: 10.5815x; 1.4889x over previous
import functools

import jax
import jax.numpy as jnp
from jax import lax
from jax.experimental import pallas as pl
from jax.experimental.pallas import tpu as pltpu
from jax.experimental.pallas import tpu_sc as plsc

N_NODES = 10000
D = 128
N_EDGES = 320000

NC = 2
NS = 16
NW = NC * NS

CHUNK = 128
CHUNKS_PER_W = 80
EDGES_PER_W = CHUNK * CHUNKS_PER_W
E_PAD = NW * EDGES_PER_W
N_PAD = 10240
ROWS_PER_TILE = N_PAD // NS


def _matmul_body(x_ref, wt_ref, o_ref):
    o_ref[...] = jnp.dot(x_ref[...], wt_ref[...],
                         preferred_element_type=jnp.float32)


def _combine_body(a_ref, b_ref, alpha_ref, o_ref):
    s = a_ref[...] + b_ref[...]
    o_ref[...] = jnp.where(s >= 0, s, alpha_ref[0, 0] * s)


def _sc_scatter_body(fts_hbm, src_hbm, dst_hbm, w_hbm, out_hbm,
                     w_v, rows0, rows1, src_d, dst_d, acc,
                     gsem0, gsem1, ssem0, ssem1,
                     xsem0, xsem1, dsem0, dsem1):
    cid = lax.axis_index("c")
    sid = lax.axis_index("s")
    wid = sid * NC + cid

    pltpu.sync_copy(w_hbm.at[wid], w_v)

    z = jnp.zeros((16,), jnp.float32)

    def zrow(r, carry):
        for c in range(8):
            rows0[r, pl.ds(16 * c, 16)] = z
        return carry

    lax.fori_loop(0, CHUNK, zrow, 0)
    base = sid * ROWS_PER_TILE
    for i in range(ROWS_PER_TILE // CHUNK):
        pltpu.sync_copy(rows0,
                        acc.at[pl.ds(base + i * CHUNK, CHUNK)])
    plsc.subcore_barrier()

    def scale(rows, j):
        def mgroup(g, c2):
            wvec = w_v[j, pl.ds(16 * g, 16)]
            for l in range(16):
                ws = wvec[l]
                r = 16 * g + l
                for c in range(8):
                    sl = rows[r, pl.ds(16 * c, 16)]
                    rows[r, pl.ds(16 * c, 16)] = sl * ws
            return c2

        lax.fori_loop(0, CHUNK // 16, mgroup, 0)

    def idx_fetch(j, b):
        xs = xsem0 if b == 0 else xsem1
        ds = dsem0 if b == 0 else dsem1
        sd = src_d.at[b] if b == 0 else src_d.at[1]
        dd = dst_d.at[b] if b == 0 else dst_d.at[1]
        pltpu.async_copy(src_hbm.at[wid, j], sd, xs)
        pltpu.async_copy(dst_hbm.at[wid, j], dd, ds)

    idx_fetch(0, 0)
    idx_fetch(1, 1)
    pltpu.make_async_copy(src_hbm.at[wid, 0], src_d.at[0], xsem0).wait()
    pltpu.async_copy(fts_hbm.at[src_d.at[0]], rows0, gsem0)

    def super_iter(g, carry):
        for b in range(2):
            j = 2 * g + b
            o = 1 - b
            rb, ro = (rows0, rows1) if b == 0 else (rows1, rows0)
            sg_b, sg_o = (gsem0, gsem1) if b == 0 else (gsem1, gsem0)
            ss_b, ss_o = (ssem0, ssem1) if b == 0 else (ssem1, ssem0)
            sx_b, sx_o = (xsem0, xsem1) if b == 0 else (xsem1, xsem0)
            sd_b, sd_o = (dsem0, dsem1) if b == 0 else (dsem1, dsem0)

            pltpu.make_async_copy(
                fts_hbm.at[src_d.at[b]], rb, sg_b).wait()

            @pl.when(j >= 1)
            def _():
                pltpu.make_async_copy(
                    ro, acc.at[dst_d.at[o]], ss_o).wait()

            @pl.when(j + 1 < CHUNKS_PER_W)
            def _():
                pltpu.make_async_copy(
                    src_hbm.at[wid, j + 1], src_d.at[o], sx_o).wait()
                pltpu.async_copy(fts_hbm.at[src_d.at[o]], ro, sg_o)

            scale(rb, j)

            pltpu.make_async_copy(
                dst_hbm.at[wid, j], dst_d.at[b], sd_b).wait()
            pltpu.async_copy(rb, acc.at[dst_d.at[b]], ss_b, add=True)

            @pl.when(j + 2 < CHUNKS_PER_W)
            def _():
                pltpu.async_copy(src_hbm.at[wid, j + 2], src_d.at[b], sx_b)
                pltpu.async_copy(dst_hbm.at[wid, j + 2], dst_d.at[b], sd_b)
        return carry

    lax.fori_loop(0, CHUNKS_PER_W // 2, super_iter, 0)
    pltpu.make_async_copy(
        rows1, acc.at[dst_d.at[1]], ssem1).wait()
    plsc.subcore_barrier()

    pltpu.sync_copy(acc.at[pl.ds(base, ROWS_PER_TILE)],
                    out_hbm.at[cid, pl.ds(base, ROWS_PER_TILE)])


_sc_scatter = functools.partial(
    pl.kernel,
    out_type=jax.ShapeDtypeStruct((NC, N_PAD, D), jnp.float32),
    mesh=plsc.VectorSubcoreMesh(core_axis_name="c", subcore_axis_name="s"),
    scratch_types=[
        pltpu.VMEM((CHUNKS_PER_W, CHUNK), jnp.float32),
        pltpu.VMEM((CHUNK, D), jnp.float32),
        pltpu.VMEM((CHUNK, D), jnp.float32),
        pltpu.VMEM((2, CHUNK), jnp.int32),
        pltpu.VMEM((2, CHUNK), jnp.int32),
        pltpu.VMEM_SHARED((N_PAD, D), jnp.float32),
        pltpu.SemaphoreType.DMA,
        pltpu.SemaphoreType.DMA,
        pltpu.SemaphoreType.DMA,
        pltpu.SemaphoreType.DMA,
        pltpu.SemaphoreType.DMA,
        pltpu.SemaphoreType.DMA,
        pltpu.SemaphoreType.DMA,
        pltpu.SemaphoreType.DMA,
    ],
)(_sc_scatter_body)


def kernel(seq, edge_index, edge_weight, W, alpha):
    wt = W.T
    fts = pl.pallas_call(
        _matmul_body,
        grid=(10,),
        in_specs=[
            pl.BlockSpec((N_NODES // 10, D), lambda i: (i, 0)),
            pl.BlockSpec((D, D), lambda i: (0, 0)),
        ],
        out_specs=pl.BlockSpec((N_NODES // 10, D), lambda i: (i, 0)),
        out_shape=jax.ShapeDtypeStruct((N_NODES, D), jnp.float32),
    )(seq, wt)

    pad = E_PAD - N_EDGES
    pad_rows = (jnp.arange(pad, dtype=jnp.int32) % N_NODES)
    dst = jnp.concatenate([edge_index[0], pad_rows])
    src = jnp.concatenate([edge_index[1], pad_rows])
    w = jnp.concatenate([edge_weight, jnp.zeros((pad,), jnp.float32)])
    src3 = src.reshape(NW, CHUNKS_PER_W, CHUNK)
    dst3 = dst.reshape(NW, CHUNKS_PER_W, CHUNK)
    w3 = w.reshape(NW, CHUNKS_PER_W, CHUNK)

    partials = _sc_scatter(fts, src3, dst3, w3)

    alpha2 = jnp.asarray(alpha, jnp.float32).reshape(1, 1)
    out = pl.pallas_call(
        _combine_body,
        grid=(10,),
        in_specs=[
            pl.BlockSpec((N_NODES // 10, D), lambda i: (i, 0)),
            pl.BlockSpec((N_NODES // 10, D), lambda i: (i, 0)),
            pl.BlockSpec((1, 1), lambda i: (0, 0)),
        ],
        out_specs=pl.BlockSpec((N_NODES // 10, D), lambda i: (i, 0)),
        out_shape=jax.ShapeDtypeStruct((N_NODES, D), jnp.float32),
    )(partials[0], partials[1], alpha2)
    return out


# speedup vs baseline: 10.8554x; 1.0259x over previous
import functools

import jax
import jax.numpy as jnp
from jax import lax
from jax.experimental import pallas as pl
from jax.experimental.pallas import tpu as pltpu
from jax.experimental.pallas import tpu_sc as plsc

N_NODES = 10000
D = 128
N_EDGES = 320000

NC = 2
NS = 16
NW = NC * NS

CHUNK = 128
CHUNKS_PER_W = 80
EDGES_PER_W = CHUNK * CHUNKS_PER_W
E_PAD = NW * EDGES_PER_W

DRAIN_ROWS = 624
DRAIN_EXTRA = N_NODES - NS * DRAIN_ROWS


def _matmul_body(x_ref, wt_ref, o_ref):
    o_ref[...] = jnp.dot(x_ref[...], wt_ref[...],
                         preferred_element_type=jnp.float32)


def _combine_body(a_ref, b_ref, alpha_ref, o_ref):
    s = a_ref[0] + b_ref[0]
    o_ref[...] = jnp.where(s >= 0, s, alpha_ref[0, 0] * s)


def _sc_scatter_body(fts_hbm, src_hbm, dst_hbm, w_hbm, out_hbm,
                     rows0, rows1, rows2, src_d, dst_d, w_d, acc,
                     gsem0, gsem1, gsem2, ssem0, ssem1, ssem2,
                     xsem0, xsem1, xsem2, dsem0, dsem1, dsem2,
                     wsem0, wsem1, wsem2):
    cid = lax.axis_index("c")
    sid = lax.axis_index("s")
    wid = sid * NC + cid

    rows = (rows0, rows1, rows2)
    gsem = (gsem0, gsem1, gsem2)
    ssem = (ssem0, ssem1, ssem2)
    xsem = (xsem0, xsem1, xsem2)
    dsem = (dsem0, dsem1, dsem2)
    wsem = (wsem0, wsem1, wsem2)

    def idx_fetch(j, s):
        pltpu.async_copy(src_hbm.at[wid, j], src_d.at[s], xsem[s])
        pltpu.async_copy(dst_hbm.at[wid, j], dst_d.at[s], dsem[s])
        pltpu.async_copy(w_hbm.at[wid, j], w_d.at[s], wsem[s])

    z = jnp.zeros((16,), jnp.float32)

    def zrow(r, carry):
        for c in range(8):
            rows0[r, pl.ds(16 * c, 16)] = z
        return carry

    lax.fori_loop(0, CHUNK, zrow, 0)
    zbase = sid * 625
    for i in range(5):
        pltpu.sync_copy(rows0.at[pl.ds(0, 125)],
                        acc.at[pl.ds(zbase + i * 125, 125)])
    plsc.subcore_barrier()

    def scale(rb, ws_slot):
        def mgroup(g, c2):
            wvec = w_d[ws_slot, pl.ds(16 * g, 16)]
            for l in range(16):
                ws = wvec[l]
                r = 16 * g + l
                for c in range(8):
                    sl = rb[r, pl.ds(16 * c, 16)]
                    rb[r, pl.ds(16 * c, 16)] = sl * ws
            return c2

        lax.fori_loop(0, CHUNK // 16, mgroup, 0)

    def iteration(j, b, n, p, is_first, is_last):
        pltpu.make_async_copy(
            fts_hbm.at[src_d.at[b]], rows[b], gsem[b]).wait()

        if not is_last:
            pltpu.make_async_copy(
                src_hbm.at[wid, 0], src_d.at[n], xsem[n]).wait()
            pltpu.async_copy(fts_hbm.at[src_d.at[n]], rows[n], gsem[n])

        pltpu.make_async_copy(
            w_hbm.at[wid, 0], w_d.at[b], wsem[b]).wait()
        scale(rows[b], b)

        pltpu.make_async_copy(
            dst_hbm.at[wid, 0], dst_d.at[b], dsem[b]).wait()
        pltpu.async_copy(rows[b], acc.at[dst_d.at[b]], ssem[b], add=True)

        if not is_first:
            pltpu.make_async_copy(
                rows[p], acc.at[dst_d.at[p]], ssem[p]).wait()

        @pl.when(j < CHUNKS_PER_W - 2)
        def _():
            idx_fetch(j + 2, p)

    idx_fetch(0, 0)
    idx_fetch(1, 1)
    pltpu.make_async_copy(
        src_hbm.at[wid, 0], src_d.at[0], xsem0).wait()
    pltpu.async_copy(fts_hbm.at[src_d.at[0]], rows0, gsem0)

    iteration(0, 0, 1, 2, is_first=True, is_last=False)
    iteration(1, 1, 2, 0, is_first=False, is_last=False)

    def super_iter(g, carry):
        for b in range(3):
            j = 3 * g + 2 + b
            iteration(j, (b + 2) % 3, b, (b + 1) % 3,
                      is_first=False, is_last=False)
        return carry

    lax.fori_loop(0, 25, super_iter, 0)
    iteration(77, 2, 0, 1, is_first=False, is_last=False)
    iteration(78, 0, 1, 2, is_first=False, is_last=False)
    iteration(79, 1, 2, 0, is_first=False, is_last=True)
    pltpu.make_async_copy(rows1, acc.at[dst_d.at[1]], ssem1).wait()
    plsc.subcore_barrier()

    dbase = sid * DRAIN_ROWS
    pltpu.sync_copy(acc.at[pl.ds(dbase, DRAIN_ROWS)],
                    out_hbm.at[cid, pl.ds(dbase, DRAIN_ROWS)])

    @pl.when(sid == NS - 1)
    def _():
        pltpu.sync_copy(
            acc.at[pl.ds(NS * DRAIN_ROWS, DRAIN_EXTRA)],
            out_hbm.at[cid, pl.ds(NS * DRAIN_ROWS, DRAIN_EXTRA)])


_sc_scatter = functools.partial(
    pl.kernel,
    out_type=jax.ShapeDtypeStruct((NC, N_NODES, D), jnp.float32),
    mesh=plsc.VectorSubcoreMesh(core_axis_name="c", subcore_axis_name="s"),
    scratch_types=[
        pltpu.VMEM((CHUNK, D), jnp.float32),
        pltpu.VMEM((CHUNK, D), jnp.float32),
        pltpu.VMEM((CHUNK, D), jnp.float32),
        pltpu.VMEM((3, CHUNK), jnp.int32),
        pltpu.VMEM((3, CHUNK), jnp.int32),
        pltpu.VMEM((3, CHUNK), jnp.float32),
        pltpu.VMEM_SHARED((N_NODES, D), jnp.float32),
    ] + [pltpu.SemaphoreType.DMA] * 15,
)(_sc_scatter_body)


def kernel(seq, edge_index, edge_weight, W, alpha):
    wt = W.T
    fts = pl.pallas_call(
        _matmul_body,
        grid=(10,),
        in_specs=[
            pl.BlockSpec((N_NODES // 10, D), lambda i: (i, 0)),
            pl.BlockSpec((D, D), lambda i: (0, 0)),
        ],
        out_specs=pl.BlockSpec((N_NODES // 10, D), lambda i: (i, 0)),
        out_shape=jax.ShapeDtypeStruct((N_NODES, D), jnp.float32),
    )(seq, wt)

    pad = E_PAD - N_EDGES
    pad_rows = (jnp.arange(pad, dtype=jnp.int32) % N_NODES)
    dst = jnp.concatenate([edge_index[0], pad_rows])
    src = jnp.concatenate([edge_index[1], pad_rows])
    w = jnp.concatenate([edge_weight, jnp.zeros((pad,), jnp.float32)])
    src3 = src.reshape(NW, CHUNKS_PER_W, CHUNK)
    dst3 = dst.reshape(NW, CHUNKS_PER_W, CHUNK)
    w3 = w.reshape(NW, CHUNKS_PER_W, CHUNK)

    partials = _sc_scatter(fts, src3, dst3, w3)

    alpha2 = jnp.asarray(alpha, jnp.float32).reshape(1, 1)
    out = pl.pallas_call(
        _combine_body,
        grid=(10,),
        in_specs=[
            pl.BlockSpec((1, N_NODES // 10, D), lambda i: (0, i, 0)),
            pl.BlockSpec((1, N_NODES // 10, D), lambda i: (1, i, 0)),
            pl.BlockSpec((1, 1), lambda i: (0, 0)),
        ],
        out_specs=pl.BlockSpec((N_NODES // 10, D), lambda i: (i, 0)),
        out_shape=jax.ShapeDtypeStruct((N_NODES, D), jnp.float32),
    )(partials, partials, alpha2)
    return out
